# baseline (speedup 1.0000x reference)
import jax
import jax.numpy as jnp
from jax import lax
from jax.experimental import pallas as pl
from jax.experimental.pallas import tpu as pltpu
from jax.experimental.pallas import tpu_sc as plsc

NC = 2
NS = 16
NW = NC * NS
K = 100
BR = 1000


def _sc_degree(dst3, zeros16, ones16, *, n):
    rows = n // NS
    nchunk, k = dst3.shape[1], dst3.shape[2]
    mesh = plsc.VectorSubcoreMesh(core_axis_name="c", subcore_axis_name="s")

    def body(dst_hbm, zeros_hbm, ones_hbm, out_hbm, dst_v, ones_v, acc):
        c = lax.axis_index("c")
        s = lax.axis_index("s")
        wid = s * NC + c
        r0 = s * rows
        pltpu.sync_copy(zeros_hbm.at[pl.ds(r0, rows)], acc.at[pl.ds(r0, rows)])
        pltpu.sync_copy(dst_hbm.at[wid], dst_v)
        pltpu.sync_copy(ones_hbm, ones_v)
        plsc.subcore_barrier()

        @pl.loop(0, nchunk)
        def _(j):
            pltpu.sync_copy(ones_v, acc.at[dst_v.at[j]], add=True)

        plsc.subcore_barrier()
        pltpu.sync_copy(acc.at[pl.ds(r0, rows)], out_hbm.at[c, pl.ds(r0, rows)])

    f = pl.kernel(
        body,
        out_type=jax.ShapeDtypeStruct((NC, n, 16), jnp.float32),
        mesh=mesh,
        compiler_params=pltpu.CompilerParams(use_tc_tiling_on_sc=False),
        scratch_types=[
            pltpu.VMEM((nchunk, k), jnp.int32),
            pltpu.VMEM((k, 16), jnp.float32),
            pltpu.VMEM_SHARED((n, 16), jnp.float32),
        ],
    )
    return f(dst3, zeros16, ones16)


def _sc_aggregate(table, src3, dst3, zeros, *, n, d, k, nbuf):
    rows = n // NS
    nchunk = src3.shape[1]
    mesh = plsc.VectorSubcoreMesh(core_axis_name="c", subcore_axis_name="s")

    def body(table_hbm, src_hbm, dst_hbm, zeros_hbm, out_hbm,
             src_v, dst_v, bufs, gsems, acc):
        c = lax.axis_index("c")
        s = lax.axis_index("s")
        wid = s * NC + c
        r0 = s * rows
        pltpu.sync_copy(zeros_hbm.at[pl.ds(r0, rows)], acc.at[pl.ds(r0, rows)])
        pltpu.sync_copy(src_hbm.at[wid], src_v)
        pltpu.sync_copy(dst_hbm.at[wid], dst_v)
        plsc.subcore_barrier()

        for b in range(nbuf):
            pltpu.async_copy(table_hbm.at[src_v.at[b]], bufs[b], gsems[b])

        @pl.loop(0, nchunk, step=nbuf)
        def _(j):
            for b in range(nbuf):
                ch = j + b
                pltpu.make_async_copy(
                    table_hbm.at[src_v.at[ch]], bufs[b], gsems[b]).wait()
                pltpu.sync_copy(bufs[b], acc.at[dst_v.at[ch]], add=True)
                nxt = ch + nbuf

                @pl.when(nxt < nchunk)
                def _():
                    pltpu.async_copy(table_hbm.at[src_v.at[nxt]], bufs[b], gsems[b])

        plsc.subcore_barrier()
        pltpu.sync_copy(acc.at[pl.ds(r0, rows)], out_hbm.at[c, pl.ds(r0, rows)])

    f = pl.kernel(
        body,
        out_type=jax.ShapeDtypeStruct((NC, n, d), jnp.float32),
        mesh=mesh,
        compiler_params=pltpu.CompilerParams(use_tc_tiling_on_sc=False),
        scratch_types=[
            pltpu.VMEM((nchunk, k), jnp.int32),
            pltpu.VMEM((nchunk, k), jnp.int32),
            tuple(pltpu.VMEM((k, d), jnp.float32) for _ in range(nbuf)),
            tuple(pltpu.SemaphoreType.DMA for _ in range(nbuf)),
            pltpu.VMEM_SHARED((n, d), jnp.float32),
        ],
    )
    return f(table, src3, dst3, zeros)


def _tc_matmul(x, W1, *, n):
    dd = x.shape[1]
    h = W1.shape[1]

    def body(x_ref, w_ref, o_ref):
        o_ref[...] = jnp.dot(x_ref[...], w_ref[...],
                             preferred_element_type=jnp.float32)

    return pl.pallas_call(
        body,
        grid=(n // BR,),
        in_specs=[
            pl.BlockSpec((BR, dd), lambda i: (i, 0)),
            pl.BlockSpec((dd, h), lambda i: (0, 0)),
        ],
        out_specs=pl.BlockSpec((BR, h), lambda i: (i, 0)),
        out_shape=jax.ShapeDtypeStruct((n, h), jnp.float32),
    )(x, W1)


def _tc_scale(degp, h1, *, n):
    h = h1.shape[1]

    def body(degp_ref, h_ref, dis_ref, s_ref):
        deg = degp_ref[0, :, 0:1] + degp_ref[1, :, 0:1] + 1.0
        dis = lax.rsqrt(deg)
        dis_ref[...] = dis
        s_ref[...] = h_ref[...] * dis

    return pl.pallas_call(
        body,
        grid=(n // BR,),
        in_specs=[
            pl.BlockSpec((2, BR, 16), lambda i: (0, i, 0)),
            pl.BlockSpec((BR, h), lambda i: (i, 0)),
        ],
        out_specs=[
            pl.BlockSpec((BR, 1), lambda i: (i, 0)),
            pl.BlockSpec((BR, h), lambda i: (i, 0)),
        ],
        out_shape=[
            jax.ShapeDtypeStruct((n, 1), jnp.float32),
            jax.ShapeDtypeStruct((n, h), jnp.float32),
        ],
    )(degp, h1)


def _tc_mid(aggp, s_prev, dis2, brow, Wn, *, n):
    h = s_prev.shape[1]
    dn = Wn.shape[1]

    def body(aggp_ref, s_ref, dis_ref, b_ref, w_ref, o_ref):
        dis = dis_ref[...]
        agg = aggp_ref[0] + aggp_ref[1] + s_ref[...]
        hh = jnp.maximum(dis * agg + b_ref[...], 0.0)
        o_ref[...] = jnp.dot(hh, w_ref[...],
                             preferred_element_type=jnp.float32) * dis

    return pl.pallas_call(
        body,
        grid=(n // BR,),
        in_specs=[
            pl.BlockSpec((2, BR, h), lambda i: (0, i, 0)),
            pl.BlockSpec((BR, h), lambda i: (i, 0)),
            pl.BlockSpec((BR, 1), lambda i: (i, 0)),
            pl.BlockSpec((1, h), lambda i: (0, 0)),
            pl.BlockSpec((h, dn), lambda i: (0, 0)),
        ],
        out_specs=pl.BlockSpec((BR, dn), lambda i: (i, 0)),
        out_shape=jax.ShapeDtypeStruct((n, dn), jnp.float32),
    )(aggp, s_prev, dis2, brow, Wn)


def _tc_last(aggp, s3, dis2, brow, *, n, c_out):
    d3 = s3.shape[1]

    def body(aggp_ref, s_ref, dis_ref, b_ref, o_ref):
        agg = aggp_ref[0] + aggp_ref[1] + s_ref[...]
        o_ref[...] = (dis_ref[...] * agg + b_ref[...])[:, :c_out]

    return pl.pallas_call(
        body,
        grid=(n // BR,),
        in_specs=[
            pl.BlockSpec((2, BR, d3), lambda i: (0, i, 0)),
            pl.BlockSpec((BR, d3), lambda i: (i, 0)),
            pl.BlockSpec((BR, 1), lambda i: (i, 0)),
            pl.BlockSpec((1, d3), lambda i: (0, 0)),
        ],
        out_specs=pl.BlockSpec((BR, c_out), lambda i: (i, 0)),
        out_shape=jax.ShapeDtypeStruct((n, c_out), jnp.float32),
    )(aggp, s3, dis2, brow)


def kernel(x, edge_index, W1, b1, W2, b2, W3, b3):
    n, dd = x.shape
    e = edge_index.shape[1]
    h = W1.shape[1]
    c_out = W3.shape[1]
    d3 = 48
    kh, nbufh = 100, 2
    k3, nbuf3 = 100, 2
    assert e % (NW * kh) == 0 and e % (NW * k3) == 0
    assert n % NS == 0 and n % BR == 0

    src, dst = edge_index[0], edge_index[1]
    srch = src.reshape(NW, e // (NW * kh), kh)
    dsth = dst.reshape(NW, e // (NW * kh), kh)
    src3 = src.reshape(NW, e // (NW * k3), k3)
    dst3 = dst.reshape(NW, e // (NW * k3), k3)
    zeros_h = jnp.zeros((n, h), jnp.float32)
    zeros_3 = jnp.zeros((n, d3), jnp.float32)
    zeros_16 = jnp.zeros((n, 16), jnp.float32)
    ones_16 = jnp.ones((k3, 16), jnp.float32)
    W3p = jnp.pad(W3, ((0, 0), (0, d3 - c_out)))
    b1r = b1.reshape(1, h)
    b2r = b2.reshape(1, h)
    b3r = jnp.pad(b3, (0, d3 - c_out)).reshape(1, d3)

    degp = _sc_degree(dst3, zeros_16, ones_16, n=n)
    h1 = _tc_matmul(x, W1, n=n)
    dis2, s1 = _tc_scale(degp, h1, n=n)
    agg1 = _sc_aggregate(s1, srch, dsth, zeros_h, n=n, d=h, k=kh, nbuf=nbufh)
    s2 = _tc_mid(agg1, s1, dis2, b1r, W2, n=n)
    agg2 = _sc_aggregate(s2, srch, dsth, zeros_h, n=n, d=h, k=kh, nbuf=nbufh)
    s3 = _tc_mid(agg2, s2, dis2, b2r, W3p, n=n)
    agg3 = _sc_aggregate(s3, src3, dst3, zeros_3, n=n, d=d3, k=k3, nbuf=nbuf3)
    return _tc_last(agg3, s3, dis2, b3r, n=n, c_out=c_out)


# speedup vs baseline: 1.0412x; 1.0412x over previous
import jax
import jax.numpy as jnp
from jax import lax
from jax.experimental import pallas as pl
from jax.experimental.pallas import tpu as pltpu
from jax.experimental.pallas import tpu_sc as plsc

NC = 2
NS = 16
NW = NC * NS
K = 100
BR = 1000


def _sc_degree(dst3, zeros16, ones16, *, n):
    rows = n // NS
    nchunk, k = dst3.shape[1], dst3.shape[2]
    mesh = plsc.VectorSubcoreMesh(core_axis_name="c", subcore_axis_name="s")

    def body(dst_hbm, zeros_hbm, ones_hbm, out_hbm, dst_v, ones_v, acc):
        c = lax.axis_index("c")
        s = lax.axis_index("s")
        wid = s * NC + c
        r0 = s * rows
        pltpu.sync_copy(zeros_hbm.at[pl.ds(r0, rows)], acc.at[pl.ds(r0, rows)])
        pltpu.sync_copy(dst_hbm.at[wid], dst_v)
        pltpu.sync_copy(ones_hbm, ones_v)
        plsc.subcore_barrier()

        @pl.loop(0, nchunk)
        def _(j):
            pltpu.sync_copy(ones_v, acc.at[dst_v.at[j]], add=True)

        plsc.subcore_barrier()
        pltpu.sync_copy(acc.at[pl.ds(r0, rows)], out_hbm.at[c, pl.ds(r0, rows)])

    f = pl.kernel(
        body,
        out_type=jax.ShapeDtypeStruct((NC, n, 16), jnp.float32),
        mesh=mesh,
        compiler_params=pltpu.CompilerParams(use_tc_tiling_on_sc=False),
        scratch_types=[
            pltpu.VMEM((nchunk, k), jnp.int32),
            pltpu.VMEM((k, 16), jnp.float32),
            pltpu.VMEM_SHARED((n, 16), jnp.float32),
        ],
    )
    return f(dst3, zeros16, ones16)


def _sc_aggregate(table, src3, dst3, zeros, *, n, d, k, nbuf):
    rows = n // NS
    nchunk = src3.shape[1]
    mesh = plsc.VectorSubcoreMesh(core_axis_name="c", subcore_axis_name="s")

    def body(table_hbm, src_hbm, dst_hbm, zeros_hbm, out_hbm,
             src_v, dst_v, bufs, gsems, acc):
        c = lax.axis_index("c")
        s = lax.axis_index("s")
        wid = s * NC + c
        r0 = s * rows
        pltpu.sync_copy(zeros_hbm.at[pl.ds(r0, rows)], acc.at[pl.ds(r0, rows)])
        pltpu.sync_copy(src_hbm.at[wid], src_v)
        pltpu.sync_copy(dst_hbm.at[wid], dst_v)
        plsc.subcore_barrier()

        for b in range(nbuf):
            pltpu.async_copy(table_hbm.at[src_v.at[b]], bufs[b], gsems[b])

        @pl.loop(0, nchunk, step=nbuf)
        def _(j):
            for b in range(nbuf):
                ch = j + b
                pltpu.make_async_copy(
                    table_hbm.at[src_v.at[ch]], bufs[b], gsems[b]).wait()
                pltpu.sync_copy(bufs[b], acc.at[dst_v.at[ch]], add=True)
                nxt = ch + nbuf

                @pl.when(nxt < nchunk)
                def _():
                    pltpu.async_copy(table_hbm.at[src_v.at[nxt]], bufs[b], gsems[b])

        plsc.subcore_barrier()
        pltpu.sync_copy(acc.at[pl.ds(r0, rows)], out_hbm.at[c, pl.ds(r0, rows)])

    f = pl.kernel(
        body,
        out_type=jax.ShapeDtypeStruct((NC, n, d), jnp.float32),
        mesh=mesh,
        compiler_params=pltpu.CompilerParams(use_tc_tiling_on_sc=False),
        scratch_types=[
            pltpu.VMEM((nchunk, k), jnp.int32),
            pltpu.VMEM((nchunk, k), jnp.int32),
            tuple(pltpu.VMEM((k, d), jnp.float32) for _ in range(nbuf)),
            tuple(pltpu.SemaphoreType.DMA for _ in range(nbuf)),
            pltpu.VMEM_SHARED((n, d), jnp.float32),
        ],
    )
    return f(table, src3, dst3, zeros)


def _tc_matmul(x, W1, *, n):
    dd = x.shape[1]
    h = W1.shape[1]

    def body(x_ref, w_ref, o_ref):
        o_ref[...] = jnp.dot(x_ref[...], w_ref[...],
                             preferred_element_type=jnp.float32)

    return pl.pallas_call(
        body,
        grid=(n // BR,),
        in_specs=[
            pl.BlockSpec((BR, dd), lambda i: (i, 0)),
            pl.BlockSpec((dd, h), lambda i: (0, 0)),
        ],
        out_specs=pl.BlockSpec((BR, h), lambda i: (i, 0)),
        out_shape=jax.ShapeDtypeStruct((n, h), jnp.float32),
    )(x, W1)


def _tc_scale(degp, h1, *, n):
    h = h1.shape[1]

    def body(degp_ref, h_ref, dis_ref, s_ref):
        deg = degp_ref[0, :, 0:1] + degp_ref[1, :, 0:1] + 1.0
        dis = lax.rsqrt(deg)
        dis_ref[...] = dis
        s_ref[...] = h_ref[...] * dis

    return pl.pallas_call(
        body,
        grid=(n // BR,),
        in_specs=[
            pl.BlockSpec((2, BR, 16), lambda i: (0, i, 0)),
            pl.BlockSpec((BR, h), lambda i: (i, 0)),
        ],
        out_specs=[
            pl.BlockSpec((BR, 1), lambda i: (i, 0)),
            pl.BlockSpec((BR, h), lambda i: (i, 0)),
        ],
        out_shape=[
            jax.ShapeDtypeStruct((n, 1), jnp.float32),
            jax.ShapeDtypeStruct((n, h), jnp.float32),
        ],
    )(degp, h1)


def _tc_mid(aggp, s_prev, dis2, brow, Wn, *, n):
    h = s_prev.shape[1]
    dn = Wn.shape[1]

    def body(aggp_ref, s_ref, dis_ref, b_ref, w_ref, o_ref):
        dis = dis_ref[...]
        agg = aggp_ref[0] + aggp_ref[1] + s_ref[...]
        hh = jnp.maximum(dis * agg + b_ref[...], 0.0)
        o_ref[...] = jnp.dot(hh, w_ref[...],
                             preferred_element_type=jnp.float32) * dis

    return pl.pallas_call(
        body,
        grid=(n // BR,),
        in_specs=[
            pl.BlockSpec((2, BR, h), lambda i: (0, i, 0)),
            pl.BlockSpec((BR, h), lambda i: (i, 0)),
            pl.BlockSpec((BR, 1), lambda i: (i, 0)),
            pl.BlockSpec((1, h), lambda i: (0, 0)),
            pl.BlockSpec((h, dn), lambda i: (0, 0)),
        ],
        out_specs=pl.BlockSpec((BR, dn), lambda i: (i, 0)),
        out_shape=jax.ShapeDtypeStruct((n, dn), jnp.float32),
    )(aggp, s_prev, dis2, brow, Wn)


def _tc_last(aggp, s3, dis2, brow, *, n, c_out):
    d3 = s3.shape[1]

    def body(aggp_ref, s_ref, dis_ref, b_ref, o_ref):
        agg = aggp_ref[0] + aggp_ref[1] + s_ref[...]
        o_ref[...] = (dis_ref[...] * agg + b_ref[...])[:, :c_out]

    return pl.pallas_call(
        body,
        grid=(n // BR,),
        in_specs=[
            pl.BlockSpec((2, BR, d3), lambda i: (0, i, 0)),
            pl.BlockSpec((BR, d3), lambda i: (i, 0)),
            pl.BlockSpec((BR, 1), lambda i: (i, 0)),
            pl.BlockSpec((1, d3), lambda i: (0, 0)),
        ],
        out_specs=pl.BlockSpec((BR, c_out), lambda i: (i, 0)),
        out_shape=jax.ShapeDtypeStruct((n, c_out), jnp.float32),
    )(aggp, s3, dis2, brow)


def kernel(x, edge_index, W1, b1, W2, b2, W3, b3):
    n, dd = x.shape
    e = edge_index.shape[1]
    h = W1.shape[1]
    c_out = W3.shape[1]
    d3 = 48
    kh, nbufh = 50, 4
    k3, nbuf3 = 100, 2
    assert e % (NW * kh) == 0 and e % (NW * k3) == 0
    assert n % NS == 0 and n % BR == 0

    src, dst = edge_index[0], edge_index[1]
    srch = src.reshape(NW, e // (NW * kh), kh)
    dsth = dst.reshape(NW, e // (NW * kh), kh)
    src3 = src.reshape(NW, e // (NW * k3), k3)
    dst3 = dst.reshape(NW, e // (NW * k3), k3)
    zeros_h = jnp.zeros((n, h), jnp.float32)
    zeros_3 = jnp.zeros((n, d3), jnp.float32)
    zeros_16 = jnp.zeros((n, 16), jnp.float32)
    ones_16 = jnp.ones((k3, 16), jnp.float32)
    W3p = jnp.pad(W3, ((0, 0), (0, d3 - c_out)))
    b1r = b1.reshape(1, h)
    b2r = b2.reshape(1, h)
    b3r = jnp.pad(b3, (0, d3 - c_out)).reshape(1, d3)

    degp = _sc_degree(dst3, zeros_16, ones_16, n=n)
    h1 = _tc_matmul(x, W1, n=n)
    dis2, s1 = _tc_scale(degp, h1, n=n)
    agg1 = _sc_aggregate(s1, srch, dsth, zeros_h, n=n, d=h, k=kh, nbuf=nbufh)
    s2 = _tc_mid(agg1, s1, dis2, b1r, W2, n=n)
    agg2 = _sc_aggregate(s2, srch, dsth, zeros_h, n=n, d=h, k=kh, nbuf=nbufh)
    s3 = _tc_mid(agg2, s2, dis2, b2r, W3p, n=n)
    agg3 = _sc_aggregate(s3, src3, dst3, zeros_3, n=n, d=d3, k=k3, nbuf=nbuf3)
    return _tc_last(agg3, s3, dis2, b3r, n=n, c_out=c_out)


# speedup vs baseline: 1.0723x; 1.0299x over previous
import jax
import jax.numpy as jnp
from jax import lax
from jax.experimental import pallas as pl
from jax.experimental.pallas import tpu as pltpu
from jax.experimental.pallas import tpu_sc as plsc

NC = 2
NS = 16
NW = NC * NS
K = 100
BR = 1000


def _sc_degree(dst3, zeros16, ones16, *, n):
    rows = n // NS
    nchunk, k = dst3.shape[1], dst3.shape[2]
    mesh = plsc.VectorSubcoreMesh(core_axis_name="c", subcore_axis_name="s")

    def body(dst_hbm, zeros_hbm, ones_hbm, out_hbm, dst_v, ones_v, acc):
        c = lax.axis_index("c")
        s = lax.axis_index("s")
        wid = s * NC + c
        r0 = s * rows
        pltpu.sync_copy(zeros_hbm.at[pl.ds(r0, rows)], acc.at[pl.ds(r0, rows)])
        pltpu.sync_copy(dst_hbm.at[wid], dst_v)
        pltpu.sync_copy(ones_hbm, ones_v)
        plsc.subcore_barrier()

        @pl.loop(0, nchunk)
        def _(j):
            pltpu.sync_copy(ones_v, acc.at[dst_v.at[j]], add=True)

        plsc.subcore_barrier()
        pltpu.sync_copy(acc.at[pl.ds(r0, rows)], out_hbm.at[c, pl.ds(r0, rows)])

    f = pl.kernel(
        body,
        out_type=jax.ShapeDtypeStruct((NC, n, 16), jnp.float32),
        mesh=mesh,
        compiler_params=pltpu.CompilerParams(use_tc_tiling_on_sc=False),
        scratch_types=[
            pltpu.VMEM((nchunk, k), jnp.int32),
            pltpu.VMEM((k, 16), jnp.float32),
            pltpu.VMEM_SHARED((n, 16), jnp.float32),
        ],
    )
    return f(dst3, zeros16, ones16)


def _sc_aggregate(table, src3, dst3, zeros, *, n, d, k, nbuf):
    rows = n // NS
    nchunk = src3.shape[1]
    mesh = plsc.VectorSubcoreMesh(core_axis_name="c", subcore_axis_name="s")

    def body(table_hbm, src_hbm, dst_hbm, zeros_hbm, out_hbm,
             src_v, dst_v, bufs, gsems, acc):
        c = lax.axis_index("c")
        s = lax.axis_index("s")
        wid = s * NC + c
        r0 = s * rows
        pltpu.sync_copy(zeros_hbm.at[pl.ds(r0, rows)], acc.at[pl.ds(r0, rows)])
        pltpu.sync_copy(src_hbm.at[wid], src_v)
        pltpu.sync_copy(dst_hbm.at[wid], dst_v)
        plsc.subcore_barrier()

        for b in range(nbuf):
            pltpu.async_copy(table_hbm.at[src_v.at[b]], bufs[b], gsems[b])

        @pl.loop(0, nchunk, step=nbuf)
        def _(j):
            for b in range(nbuf):
                ch = j + b
                pltpu.make_async_copy(
                    table_hbm.at[src_v.at[ch]], bufs[b], gsems[b]).wait()
                pltpu.sync_copy(bufs[b], acc.at[dst_v.at[ch]], add=True)
                nxt = ch + nbuf

                @pl.when(nxt < nchunk)
                def _():
                    pltpu.async_copy(table_hbm.at[src_v.at[nxt]], bufs[b], gsems[b])

        plsc.subcore_barrier()
        pltpu.sync_copy(acc.at[pl.ds(r0, rows)], out_hbm.at[c, pl.ds(r0, rows)])

    f = pl.kernel(
        body,
        out_type=jax.ShapeDtypeStruct((NC, n, d), jnp.float32),
        mesh=mesh,
        compiler_params=pltpu.CompilerParams(use_tc_tiling_on_sc=False),
        scratch_types=[
            pltpu.VMEM((nchunk, k), jnp.int32),
            pltpu.VMEM((nchunk, k), jnp.int32),
            tuple(pltpu.VMEM((k, d), jnp.float32) for _ in range(nbuf)),
            tuple(pltpu.SemaphoreType.DMA for _ in range(nbuf)),
            pltpu.VMEM_SHARED((n, d), jnp.float32),
        ],
    )
    return f(table, src3, dst3, zeros)


def _tc_first(degp, x, W1, *, n):
    dd = x.shape[1]
    h = W1.shape[1]

    def body(degp_ref, x_ref, w_ref, dis_ref, s_ref):
        deg = degp_ref[0, :, 0:1] + degp_ref[1, :, 0:1] + 1.0
        dis = lax.rsqrt(deg)
        dis_ref[...] = dis
        hh = jnp.dot(x_ref[...], w_ref[...], preferred_element_type=jnp.float32)
        s_ref[...] = hh * dis

    return pl.pallas_call(
        body,
        grid=(n // BR,),
        in_specs=[
            pl.BlockSpec((2, BR, 16), lambda i: (0, i, 0)),
            pl.BlockSpec((BR, dd), lambda i: (i, 0)),
            pl.BlockSpec((dd, h), lambda i: (0, 0)),
        ],
        out_specs=[
            pl.BlockSpec((BR, 1), lambda i: (i, 0)),
            pl.BlockSpec((BR, h), lambda i: (i, 0)),
        ],
        out_shape=[
            jax.ShapeDtypeStruct((n, 1), jnp.float32),
            jax.ShapeDtypeStruct((n, h), jnp.float32),
        ],
    )(degp, x, W1)


def _tc_mid(aggp, s_prev, dis2, brow, Wn, *, n):
    h = s_prev.shape[1]
    dn = Wn.shape[1]

    def body(aggp_ref, s_ref, dis_ref, b_ref, w_ref, o_ref):
        dis = dis_ref[...]
        agg = aggp_ref[0] + aggp_ref[1] + s_ref[...]
        hh = jnp.maximum(dis * agg + b_ref[...], 0.0)
        o_ref[...] = jnp.dot(hh, w_ref[...],
                             preferred_element_type=jnp.float32) * dis

    return pl.pallas_call(
        body,
        grid=(n // BR,),
        in_specs=[
            pl.BlockSpec((2, BR, h), lambda i: (0, i, 0)),
            pl.BlockSpec((BR, h), lambda i: (i, 0)),
            pl.BlockSpec((BR, 1), lambda i: (i, 0)),
            pl.BlockSpec((1, h), lambda i: (0, 0)),
            pl.BlockSpec((h, dn), lambda i: (0, 0)),
        ],
        out_specs=pl.BlockSpec((BR, dn), lambda i: (i, 0)),
        out_shape=jax.ShapeDtypeStruct((n, dn), jnp.float32),
    )(aggp, s_prev, dis2, brow, Wn)


def _tc_last(aggp, s3, dis2, brow, *, n, c_out):
    d3 = s3.shape[1]

    def body(aggp_ref, s_ref, dis_ref, b_ref, o_ref):
        agg = aggp_ref[0] + aggp_ref[1] + s_ref[...]
        o_ref[...] = (dis_ref[...] * agg + b_ref[...])[:, :c_out]

    return pl.pallas_call(
        body,
        grid=(n // BR,),
        in_specs=[
            pl.BlockSpec((2, BR, d3), lambda i: (0, i, 0)),
            pl.BlockSpec((BR, d3), lambda i: (i, 0)),
            pl.BlockSpec((BR, 1), lambda i: (i, 0)),
            pl.BlockSpec((1, d3), lambda i: (0, 0)),
        ],
        out_specs=pl.BlockSpec((BR, c_out), lambda i: (i, 0)),
        out_shape=jax.ShapeDtypeStruct((n, c_out), jnp.float32),
    )(aggp, s3, dis2, brow)


def kernel(x, edge_index, W1, b1, W2, b2, W3, b3):
    n, dd = x.shape
    e = edge_index.shape[1]
    h = W1.shape[1]
    c_out = W3.shape[1]
    d3 = 48
    kh, nbufh = 50, 4
    k3, nbuf3 = 50, 4
    assert e % (NW * kh) == 0 and e % (NW * k3) == 0
    assert n % NS == 0 and n % BR == 0

    src, dst = edge_index[0], edge_index[1]
    srch = src.reshape(NW, e // (NW * kh), kh)
    dsth = dst.reshape(NW, e // (NW * kh), kh)
    src3 = src.reshape(NW, e // (NW * k3), k3)
    dst3 = dst.reshape(NW, e // (NW * k3), k3)
    zeros_h = jnp.zeros((n, h), jnp.float32)
    zeros_3 = jnp.zeros((n, d3), jnp.float32)
    zeros_16 = jnp.zeros((n, 16), jnp.float32)
    ones_16 = jnp.ones((k3, 16), jnp.float32)
    W3p = jnp.pad(W3, ((0, 0), (0, d3 - c_out)))
    b1r = b1.reshape(1, h)
    b2r = b2.reshape(1, h)
    b3r = jnp.pad(b3, (0, d3 - c_out)).reshape(1, d3)

    degp = _sc_degree(dst3, zeros_16, ones_16, n=n)
    dis2, s1 = _tc_first(degp, x, W1, n=n)
    agg1 = _sc_aggregate(s1, srch, dsth, zeros_h, n=n, d=h, k=kh, nbuf=nbufh)
    s2 = _tc_mid(agg1, s1, dis2, b1r, W2, n=n)
    agg2 = _sc_aggregate(s2, srch, dsth, zeros_h, n=n, d=h, k=kh, nbuf=nbufh)
    s3 = _tc_mid(agg2, s2, dis2, b2r, W3p, n=n)
    agg3 = _sc_aggregate(s3, src3, dst3, zeros_3, n=n, d=d3, k=k3, nbuf=nbuf3)
    return _tc_last(agg3, s3, dis2, b3r, n=n, c_out=c_out)


# speedup vs baseline: 1.1382x; 1.0615x over previous
import jax
import jax.numpy as jnp
from jax import lax
from jax.experimental import pallas as pl
from jax.experimental.pallas import tpu as pltpu
from jax.experimental.pallas import tpu_sc as plsc

NC = 2
NS = 16
NW = NC * NS
K = 100
BR = 1000


def _sc_degree(e4, zeros16, ones16, *, n):
    rows = n // NS
    nchunk, k = e4.shape[2], e4.shape[3]
    mesh = plsc.VectorSubcoreMesh(core_axis_name="c", subcore_axis_name="s")

    def body(e_hbm, zeros_hbm, ones_hbm, out_hbm, dst_v, ones_v, acc):
        c = lax.axis_index("c")
        s = lax.axis_index("s")
        wid = s * NC + c
        r0 = s * rows
        pltpu.sync_copy(zeros_hbm.at[pl.ds(r0, rows)], acc.at[pl.ds(r0, rows)])
        pltpu.sync_copy(e_hbm.at[1, wid], dst_v)
        pltpu.sync_copy(ones_hbm, ones_v)
        plsc.subcore_barrier()

        @pl.loop(0, nchunk)
        def _(j):
            pltpu.sync_copy(ones_v, acc.at[dst_v.at[j]], add=True)

        plsc.subcore_barrier()
        pltpu.sync_copy(acc.at[pl.ds(r0, rows)], out_hbm.at[c, pl.ds(r0, rows)])

    f = pl.kernel(
        body,
        out_type=jax.ShapeDtypeStruct((NC, n, 16), jnp.float32),
        mesh=mesh,
        compiler_params=pltpu.CompilerParams(use_tc_tiling_on_sc=False),
        scratch_types=[
            pltpu.VMEM((nchunk, k), jnp.int32),
            pltpu.VMEM((k, 16), jnp.float32),
            pltpu.VMEM_SHARED((n, 16), jnp.float32),
        ],
    )
    return f(e4, zeros16, ones16)


def _sc_aggregate(table, e4, zeros, *, n, d, nbuf):
    rows = n // NS
    nchunk, k = e4.shape[2], e4.shape[3]
    mesh = plsc.VectorSubcoreMesh(core_axis_name="c", subcore_axis_name="s")

    def body(table_hbm, e_hbm, zeros_hbm, out_hbm,
             src_v, dst_v, bufs, gsems, acc):
        c = lax.axis_index("c")
        s = lax.axis_index("s")
        wid = s * NC + c
        r0 = s * rows
        pltpu.sync_copy(zeros_hbm.at[pl.ds(r0, rows)], acc.at[pl.ds(r0, rows)])
        pltpu.sync_copy(e_hbm.at[0, wid], src_v)
        pltpu.sync_copy(e_hbm.at[1, wid], dst_v)
        plsc.subcore_barrier()

        for b in range(nbuf):
            pltpu.async_copy(table_hbm.at[src_v.at[b]], bufs[b], gsems[b])

        @pl.loop(0, nchunk, step=nbuf)
        def _(j):
            for b in range(nbuf):
                ch = j + b
                pltpu.make_async_copy(
                    table_hbm.at[src_v.at[ch]], bufs[b], gsems[b]).wait()
                pltpu.sync_copy(bufs[b], acc.at[dst_v.at[ch]], add=True)
                nxt = ch + nbuf

                @pl.when(nxt < nchunk)
                def _():
                    pltpu.async_copy(table_hbm.at[src_v.at[nxt]], bufs[b], gsems[b])

        plsc.subcore_barrier()
        pltpu.sync_copy(acc.at[pl.ds(r0, rows)], out_hbm.at[c, pl.ds(r0, rows)])

    f = pl.kernel(
        body,
        out_type=jax.ShapeDtypeStruct((NC, n, d), jnp.float32),
        mesh=mesh,
        compiler_params=pltpu.CompilerParams(use_tc_tiling_on_sc=False),
        scratch_types=[
            pltpu.VMEM((nchunk, k), jnp.int32),
            pltpu.VMEM((nchunk, k), jnp.int32),
            tuple(pltpu.VMEM((k, d), jnp.float32) for _ in range(nbuf)),
            tuple(pltpu.SemaphoreType.DMA for _ in range(nbuf)),
            pltpu.VMEM_SHARED((n, d), jnp.float32),
        ],
    )
    return f(table, e4, zeros)


def _tc_first(degp, x, W1, *, n):
    dd = x.shape[1]
    h = W1.shape[1]

    def body(degp_ref, x_ref, w_ref, dis_ref, s_ref):
        deg = degp_ref[0, :, 0:1] + degp_ref[1, :, 0:1] + 1.0
        dis = lax.rsqrt(deg)
        dis_ref[...] = dis
        hh = jnp.dot(x_ref[...], w_ref[...], preferred_element_type=jnp.float32)
        s_ref[...] = hh * dis

    return pl.pallas_call(
        body,
        grid=(n // BR,),
        in_specs=[
            pl.BlockSpec((2, BR, 16), lambda i: (0, i, 0)),
            pl.BlockSpec((BR, dd), lambda i: (i, 0)),
            pl.BlockSpec((dd, h), lambda i: (0, 0)),
        ],
        out_specs=[
            pl.BlockSpec((BR, 1), lambda i: (i, 0)),
            pl.BlockSpec((BR, h), lambda i: (i, 0)),
        ],
        out_shape=[
            jax.ShapeDtypeStruct((n, 1), jnp.float32),
            jax.ShapeDtypeStruct((n, h), jnp.float32),
        ],
    )(degp, x, W1)


def _tc_mid(aggp, s_prev, dis2, brow, Wn, *, n):
    h = s_prev.shape[1]
    dn = Wn.shape[1]

    def body(aggp_ref, s_ref, dis_ref, b_ref, w_ref, o_ref):
        dis = dis_ref[...]
        agg = aggp_ref[0] + aggp_ref[1] + s_ref[...]
        hh = jnp.maximum(dis * agg + b_ref[...], 0.0)
        o_ref[...] = jnp.dot(hh, w_ref[...],
                             preferred_element_type=jnp.float32) * dis

    return pl.pallas_call(
        body,
        grid=(n // BR,),
        in_specs=[
            pl.BlockSpec((2, BR, h), lambda i: (0, i, 0)),
            pl.BlockSpec((BR, h), lambda i: (i, 0)),
            pl.BlockSpec((BR, 1), lambda i: (i, 0)),
            pl.BlockSpec((1, h), lambda i: (0, 0)),
            pl.BlockSpec((h, dn), lambda i: (0, 0)),
        ],
        out_specs=pl.BlockSpec((BR, dn), lambda i: (i, 0)),
        out_shape=jax.ShapeDtypeStruct((n, dn), jnp.float32),
    )(aggp, s_prev, dis2, brow, Wn)


def _tc_last(aggp, s3, dis2, brow, *, n, c_out):
    d3 = s3.shape[1]

    def body(aggp_ref, s_ref, dis_ref, b_ref, o_ref):
        agg = aggp_ref[0] + aggp_ref[1] + s_ref[...]
        o_ref[...] = (dis_ref[...] * agg + b_ref[...])[:, :c_out]

    return pl.pallas_call(
        body,
        grid=(n // BR,),
        in_specs=[
            pl.BlockSpec((2, BR, d3), lambda i: (0, i, 0)),
            pl.BlockSpec((BR, d3), lambda i: (i, 0)),
            pl.BlockSpec((BR, 1), lambda i: (i, 0)),
            pl.BlockSpec((1, d3), lambda i: (0, 0)),
        ],
        out_specs=pl.BlockSpec((BR, c_out), lambda i: (i, 0)),
        out_shape=jax.ShapeDtypeStruct((n, c_out), jnp.float32),
    )(aggp, s3, dis2, brow)


def kernel(x, edge_index, W1, b1, W2, b2, W3, b3):
    n, dd = x.shape
    e = edge_index.shape[1]
    h = W1.shape[1]
    c_out = W3.shape[1]
    d3 = 48
    k, nbuf = 40, 5
    assert e % (NW * k) == 0 and n % NS == 0 and n % BR == 0
    nchunk = e // (NW * k)
    assert nchunk % nbuf == 0

    e4 = edge_index.reshape(2, NW, nchunk, k)
    zeros_h = jnp.zeros((n, h), jnp.float32)
    zeros_3 = jnp.zeros((n, d3), jnp.float32)
    zeros_16 = jnp.zeros((n, 16), jnp.float32)
    ones_16 = jnp.ones((k, 16), jnp.float32)
    W3p = jnp.pad(W3, ((0, 0), (0, d3 - c_out)))
    b1r = b1.reshape(1, h)
    b2r = b2.reshape(1, h)
    b3r = jnp.pad(b3, (0, d3 - c_out)).reshape(1, d3)

    degp = _sc_degree(e4, zeros_16, ones_16, n=n)
    dis2, s1 = _tc_first(degp, x, W1, n=n)
    agg1 = _sc_aggregate(s1, e4, zeros_h, n=n, d=h, nbuf=nbuf)
    s2 = _tc_mid(agg1, s1, dis2, b1r, W2, n=n)
    agg2 = _sc_aggregate(s2, e4, zeros_h, n=n, d=h, nbuf=nbuf)
    s3 = _tc_mid(agg2, s2, dis2, b2r, W3p, n=n)
    agg3 = _sc_aggregate(s3, e4, zeros_3, n=n, d=d3, nbuf=nbuf)
    return _tc_last(agg3, s3, dis2, b3r, n=n, c_out=c_out)


# speedup vs baseline: 1.2123x; 1.0651x over previous
import jax
import jax.numpy as jnp
from jax import lax
from jax.experimental import pallas as pl
from jax.experimental.pallas import tpu as pltpu
from jax.experimental.pallas import tpu_sc as plsc

NC = 2
NS = 16
NW = NC * NS
K = 100
BR = 2000


def _sc_degree(e4, zeros16, ones16, *, n):
    rows = n // NS
    nchunk, k = e4.shape[2], e4.shape[3]
    mesh = plsc.VectorSubcoreMesh(core_axis_name="c", subcore_axis_name="s")

    def body(e_hbm, zeros_hbm, ones_hbm, out_hbm, dst_v, ones_v, sem, acc):
        c = lax.axis_index("c")
        s = lax.axis_index("s")
        wid = s * NC + c
        r0 = s * rows
        pltpu.sync_copy(zeros_hbm.at[pl.ds(r0, rows)], acc.at[pl.ds(r0, rows)])
        pltpu.sync_copy(e_hbm.at[1, wid], dst_v)
        pltpu.sync_copy(ones_hbm, ones_v)
        plsc.subcore_barrier()

        nb = 10
        for t in range(nb):
            pltpu.async_copy(ones_v, acc.at[dst_v.at[t]], sem, add=True)

        @pl.loop(0, nchunk - nb, step=nb)
        def _(j):
            for t in range(nb):
                pltpu.async_copy(ones_v, acc.at[dst_v.at[j + nb + t]], sem, add=True)
            for t in range(nb):
                pltpu.make_async_copy(ones_v, acc.at[dst_v.at[0]], sem).wait()

        for t in range(nb):
            pltpu.make_async_copy(ones_v, acc.at[dst_v.at[0]], sem).wait()

        plsc.subcore_barrier()
        pltpu.sync_copy(acc.at[pl.ds(r0, rows)], out_hbm.at[c, pl.ds(r0, rows)])

    f = pl.kernel(
        body,
        out_type=jax.ShapeDtypeStruct((NC, n, 16), jnp.float32),
        mesh=mesh,
        compiler_params=pltpu.CompilerParams(use_tc_tiling_on_sc=False),
        scratch_types=[
            pltpu.VMEM((nchunk, k), jnp.int32),
            pltpu.VMEM((k, 16), jnp.float32),
            pltpu.SemaphoreType.DMA,
            pltpu.VMEM_SHARED((n, 16), jnp.float32),
        ],
    )
    return f(e4, zeros16, ones16)


def _sc_aggregate(table, e4, zeros, *, n, d, nbuf):
    rows = n // NS
    nchunk, k = e4.shape[2], e4.shape[3]
    mesh = plsc.VectorSubcoreMesh(core_axis_name="c", subcore_axis_name="s")

    def body(table_hbm, e_hbm, zeros_hbm, out_hbm,
             src_v, dst_v, bufs, gsems, acc):
        c = lax.axis_index("c")
        s = lax.axis_index("s")
        wid = s * NC + c
        r0 = s * rows
        pltpu.sync_copy(zeros_hbm.at[pl.ds(r0, rows)], acc.at[pl.ds(r0, rows)])
        pltpu.sync_copy(e_hbm.at[0, wid], src_v)
        pltpu.sync_copy(e_hbm.at[1, wid], dst_v)
        plsc.subcore_barrier()

        for b in range(nbuf):
            pltpu.async_copy(table_hbm.at[src_v.at[b]], bufs[b], gsems[b])

        @pl.loop(0, nchunk, step=nbuf)
        def _(j):
            for b in range(nbuf):
                ch = j + b
                pltpu.make_async_copy(
                    table_hbm.at[src_v.at[ch]], bufs[b], gsems[b]).wait()
                pltpu.sync_copy(bufs[b], acc.at[dst_v.at[ch]], add=True)
                nxt = ch + nbuf

                @pl.when(nxt < nchunk)
                def _():
                    pltpu.async_copy(table_hbm.at[src_v.at[nxt]], bufs[b], gsems[b])

        plsc.subcore_barrier()
        pltpu.sync_copy(acc.at[pl.ds(r0, rows)], out_hbm.at[c, pl.ds(r0, rows)])

    f = pl.kernel(
        body,
        out_type=jax.ShapeDtypeStruct((NC, n, d), jnp.float32),
        mesh=mesh,
        compiler_params=pltpu.CompilerParams(use_tc_tiling_on_sc=False),
        scratch_types=[
            pltpu.VMEM((nchunk, k), jnp.int32),
            pltpu.VMEM((nchunk, k), jnp.int32),
            tuple(pltpu.VMEM((k, d), jnp.float32) for _ in range(nbuf)),
            tuple(pltpu.SemaphoreType.DMA for _ in range(nbuf)),
            pltpu.VMEM_SHARED((n, d), jnp.float32),
        ],
    )
    return f(table, e4, zeros)


def _tc_first(degp, x, W1, *, n):
    dd = x.shape[1]
    h = W1.shape[1]

    def body(degp_ref, x_ref, w_ref, dis_ref, s_ref):
        deg = degp_ref[0, :, 0:1] + degp_ref[1, :, 0:1] + 1.0
        dis = lax.rsqrt(deg)
        dis_ref[...] = dis
        hh = jnp.dot(x_ref[...], w_ref[...], preferred_element_type=jnp.float32)
        s_ref[...] = hh * dis

    return pl.pallas_call(
        body,
        grid=(n // BR,),
        in_specs=[
            pl.BlockSpec((2, BR, 16), lambda i: (0, i, 0)),
            pl.BlockSpec((BR, dd), lambda i: (i, 0)),
            pl.BlockSpec((dd, h), lambda i: (0, 0)),
        ],
        out_specs=[
            pl.BlockSpec((BR, 1), lambda i: (i, 0)),
            pl.BlockSpec((BR, h), lambda i: (i, 0)),
        ],
        out_shape=[
            jax.ShapeDtypeStruct((n, 1), jnp.float32),
            jax.ShapeDtypeStruct((n, h), jnp.float32),
        ],
    )(degp, x, W1)


def _tc_mid(aggp, s_prev, dis2, brow, Wn, *, n):
    h = s_prev.shape[1]
    dn = Wn.shape[1]

    def body(aggp_ref, s_ref, dis_ref, b_ref, w_ref, o_ref):
        dis = dis_ref[...]
        agg = aggp_ref[0] + aggp_ref[1] + s_ref[...]
        hh = jnp.maximum(dis * agg + b_ref[...], 0.0)
        o_ref[...] = jnp.dot(hh, w_ref[...],
                             preferred_element_type=jnp.float32) * dis

    return pl.pallas_call(
        body,
        grid=(n // BR,),
        in_specs=[
            pl.BlockSpec((2, BR, h), lambda i: (0, i, 0)),
            pl.BlockSpec((BR, h), lambda i: (i, 0)),
            pl.BlockSpec((BR, 1), lambda i: (i, 0)),
            pl.BlockSpec((1, h), lambda i: (0, 0)),
            pl.BlockSpec((h, dn), lambda i: (0, 0)),
        ],
        out_specs=pl.BlockSpec((BR, dn), lambda i: (i, 0)),
        out_shape=jax.ShapeDtypeStruct((n, dn), jnp.float32),
    )(aggp, s_prev, dis2, brow, Wn)


def _tc_last(aggp, s3, dis2, brow, *, n, c_out):
    d3 = s3.shape[1]

    def body(aggp_ref, s_ref, dis_ref, b_ref, o_ref):
        agg = aggp_ref[0] + aggp_ref[1] + s_ref[...]
        o_ref[...] = (dis_ref[...] * agg + b_ref[...])[:, :c_out]

    return pl.pallas_call(
        body,
        grid=(n // BR,),
        in_specs=[
            pl.BlockSpec((2, BR, d3), lambda i: (0, i, 0)),
            pl.BlockSpec((BR, d3), lambda i: (i, 0)),
            pl.BlockSpec((BR, 1), lambda i: (i, 0)),
            pl.BlockSpec((1, d3), lambda i: (0, 0)),
        ],
        out_specs=pl.BlockSpec((BR, c_out), lambda i: (i, 0)),
        out_shape=jax.ShapeDtypeStruct((n, c_out), jnp.float32),
    )(aggp, s3, dis2, brow)


def kernel(x, edge_index, W1, b1, W2, b2, W3, b3):
    n, dd = x.shape
    e = edge_index.shape[1]
    h = W1.shape[1]
    c_out = W3.shape[1]
    d3 = 48
    k, nbuf = 40, 5
    assert e % (NW * k) == 0 and n % NS == 0 and n % BR == 0
    nchunk = e // (NW * k)
    assert nchunk % nbuf == 0

    e4 = edge_index.reshape(2, NW, nchunk, k)
    zeros_h = jnp.zeros((n, h), jnp.float32)
    zeros_3 = jnp.zeros((n, d3), jnp.float32)
    zeros_16 = jnp.zeros((n, 16), jnp.float32)
    ones_16 = jnp.ones((k, 16), jnp.float32)
    W3p = jnp.pad(W3, ((0, 0), (0, d3 - c_out)))
    b1r = b1.reshape(1, h)
    b2r = b2.reshape(1, h)
    b3r = jnp.pad(b3, (0, d3 - c_out)).reshape(1, d3)

    degp = _sc_degree(e4, zeros_16, ones_16, n=n)
    dis2, s1 = _tc_first(degp, x, W1, n=n)
    agg1 = _sc_aggregate(s1, e4, zeros_h, n=n, d=h, nbuf=nbuf)
    s2 = _tc_mid(agg1, s1, dis2, b1r, W2, n=n)
    agg2 = _sc_aggregate(s2, e4, zeros_h, n=n, d=h, nbuf=nbuf)
    s3 = _tc_mid(agg2, s2, dis2, b2r, W3p, n=n)
    agg3 = _sc_aggregate(s3, e4, zeros_3, n=n, d=d3, nbuf=nbuf)
    return _tc_last(agg3, s3, dis2, b3r, n=n, c_out=c_out)


# speedup vs baseline: 1.2144x; 1.0017x over previous
import jax
import jax.numpy as jnp
from jax import lax
from jax.experimental import pallas as pl
from jax.experimental.pallas import tpu as pltpu
from jax.experimental.pallas import tpu_sc as plsc

NC = 2
NS = 16
NW = NC * NS
K = 100
BR = 2000


def _sc_degree(e4, zeros16, ones16, *, n):
    rows = n // NS
    nchunk, k = e4.shape[2], e4.shape[3]
    mesh = plsc.VectorSubcoreMesh(core_axis_name="c", subcore_axis_name="s")

    def body(e_hbm, zeros_hbm, ones_hbm, out_hbm, dst_v, ones_v, sem, acc):
        c = lax.axis_index("c")
        s = lax.axis_index("s")
        wid = s * NC + c
        r0 = s * rows
        pltpu.sync_copy(zeros_hbm.at[pl.ds(r0, rows)], acc.at[pl.ds(r0, rows)])
        pltpu.sync_copy(e_hbm.at[1, wid], dst_v)
        pltpu.sync_copy(ones_hbm, ones_v)
        plsc.subcore_barrier()

        nb = 10
        for t in range(nb):
            pltpu.async_copy(ones_v, acc.at[dst_v.at[t]], sem, add=True)

        @pl.loop(0, nchunk - nb, step=nb)
        def _(j):
            for t in range(nb):
                pltpu.async_copy(ones_v, acc.at[dst_v.at[j + nb + t]], sem, add=True)
            for t in range(nb):
                pltpu.make_async_copy(ones_v, acc.at[dst_v.at[0]], sem).wait()

        for t in range(nb):
            pltpu.make_async_copy(ones_v, acc.at[dst_v.at[0]], sem).wait()

        plsc.subcore_barrier()
        pltpu.sync_copy(acc.at[pl.ds(r0, rows)], out_hbm.at[c, pl.ds(r0, rows)])

    f = pl.kernel(
        body,
        out_type=jax.ShapeDtypeStruct((NC, n, 16), jnp.float32),
        mesh=mesh,
        compiler_params=pltpu.CompilerParams(use_tc_tiling_on_sc=False),
        scratch_types=[
            pltpu.VMEM((nchunk, k), jnp.int32),
            pltpu.VMEM((k, 16), jnp.float32),
            pltpu.SemaphoreType.DMA,
            pltpu.VMEM_SHARED((n, 16), jnp.float32),
        ],
    )
    return f(e4, zeros16, ones16)


def _sc_aggregate(table, e4, zeros, *, n, d, nbuf):
    rows = n // NS
    nchunk, k = e4.shape[2], e4.shape[3]
    mesh = plsc.VectorSubcoreMesh(core_axis_name="c", subcore_axis_name="s")

    def body(table_hbm, e_hbm, zeros_hbm, out_hbm,
             src_v, dst_v, bufs, gsems, ssems, acc):
        c = lax.axis_index("c")
        s = lax.axis_index("s")
        wid = s * NC + c
        r0 = s * rows
        pltpu.sync_copy(zeros_hbm.at[pl.ds(r0, rows)], acc.at[pl.ds(r0, rows)])
        pltpu.sync_copy(e_hbm.at[0, wid], src_v)
        pltpu.sync_copy(e_hbm.at[1, wid], dst_v)
        plsc.subcore_barrier()

        for b in range(nbuf):
            pltpu.async_copy(table_hbm.at[src_v.at[b]], bufs[b], gsems[b])

        @pl.loop(0, nchunk, step=nbuf)
        def _(j):
            for b in range(nbuf):
                ch = j + b
                pltpu.make_async_copy(
                    table_hbm.at[src_v.at[ch]], bufs[b], gsems[b]).wait()
                pltpu.async_copy(bufs[b], acc.at[dst_v.at[ch]], ssems[b], add=True)
                nxt = ch + nbuf

                @pl.when(nxt < nchunk)
                def _():
                    pltpu.make_async_copy(
                        bufs[b], acc.at[dst_v.at[0]], ssems[b]).wait()
                    pltpu.async_copy(table_hbm.at[src_v.at[nxt]], bufs[b], gsems[b])

        for b in range(nbuf):
            pltpu.make_async_copy(bufs[b], acc.at[dst_v.at[0]], ssems[b]).wait()
        plsc.subcore_barrier()
        pltpu.sync_copy(acc.at[pl.ds(r0, rows)], out_hbm.at[c, pl.ds(r0, rows)])

    f = pl.kernel(
        body,
        out_type=jax.ShapeDtypeStruct((NC, n, d), jnp.float32),
        mesh=mesh,
        compiler_params=pltpu.CompilerParams(use_tc_tiling_on_sc=False),
        scratch_types=[
            pltpu.VMEM((nchunk, k), jnp.int32),
            pltpu.VMEM((nchunk, k), jnp.int32),
            tuple(pltpu.VMEM((k, d), jnp.float32) for _ in range(nbuf)),
            tuple(pltpu.SemaphoreType.DMA for _ in range(nbuf)),
            tuple(pltpu.SemaphoreType.DMA for _ in range(nbuf)),
            pltpu.VMEM_SHARED((n, d), jnp.float32),
        ],
    )
    return f(table, e4, zeros)


def _tc_first(degp, x, W1, *, n):
    dd = x.shape[1]
    h = W1.shape[1]

    def body(degp_ref, x_ref, w_ref, dis_ref, s_ref):
        deg = degp_ref[0, :, 0:1] + degp_ref[1, :, 0:1] + 1.0
        dis = lax.rsqrt(deg)
        dis_ref[...] = dis
        hh = jnp.dot(x_ref[...], w_ref[...], preferred_element_type=jnp.float32)
        s_ref[...] = hh * dis

    return pl.pallas_call(
        body,
        grid=(n // BR,),
        in_specs=[
            pl.BlockSpec((2, BR, 16), lambda i: (0, i, 0)),
            pl.BlockSpec((BR, dd), lambda i: (i, 0)),
            pl.BlockSpec((dd, h), lambda i: (0, 0)),
        ],
        out_specs=[
            pl.BlockSpec((BR, 1), lambda i: (i, 0)),
            pl.BlockSpec((BR, h), lambda i: (i, 0)),
        ],
        out_shape=[
            jax.ShapeDtypeStruct((n, 1), jnp.float32),
            jax.ShapeDtypeStruct((n, h), jnp.float32),
        ],
    )(degp, x, W1)


def _tc_mid(aggp, s_prev, dis2, brow, Wn, *, n):
    h = s_prev.shape[1]
    dn = Wn.shape[1]

    def body(aggp_ref, s_ref, dis_ref, b_ref, w_ref, o_ref):
        dis = dis_ref[...]
        agg = aggp_ref[0] + aggp_ref[1] + s_ref[...]
        hh = jnp.maximum(dis * agg + b_ref[...], 0.0)
        o_ref[...] = jnp.dot(hh, w_ref[...],
                             preferred_element_type=jnp.float32) * dis

    return pl.pallas_call(
        body,
        grid=(n // BR,),
        in_specs=[
            pl.BlockSpec((2, BR, h), lambda i: (0, i, 0)),
            pl.BlockSpec((BR, h), lambda i: (i, 0)),
            pl.BlockSpec((BR, 1), lambda i: (i, 0)),
            pl.BlockSpec((1, h), lambda i: (0, 0)),
            pl.BlockSpec((h, dn), lambda i: (0, 0)),
        ],
        out_specs=pl.BlockSpec((BR, dn), lambda i: (i, 0)),
        out_shape=jax.ShapeDtypeStruct((n, dn), jnp.float32),
    )(aggp, s_prev, dis2, brow, Wn)


def _tc_last(aggp, s3, dis2, brow, *, n, c_out):
    d3 = s3.shape[1]

    def body(aggp_ref, s_ref, dis_ref, b_ref, o_ref):
        agg = aggp_ref[0] + aggp_ref[1] + s_ref[...]
        o_ref[...] = (dis_ref[...] * agg + b_ref[...])[:, :c_out]

    return pl.pallas_call(
        body,
        grid=(n // BR,),
        in_specs=[
            pl.BlockSpec((2, BR, d3), lambda i: (0, i, 0)),
            pl.BlockSpec((BR, d3), lambda i: (i, 0)),
            pl.BlockSpec((BR, 1), lambda i: (i, 0)),
            pl.BlockSpec((1, d3), lambda i: (0, 0)),
        ],
        out_specs=pl.BlockSpec((BR, c_out), lambda i: (i, 0)),
        out_shape=jax.ShapeDtypeStruct((n, c_out), jnp.float32),
    )(aggp, s3, dis2, brow)


def kernel(x, edge_index, W1, b1, W2, b2, W3, b3):
    n, dd = x.shape
    e = edge_index.shape[1]
    h = W1.shape[1]
    c_out = W3.shape[1]
    d3 = 48
    k, nbuf = 40, 5
    assert e % (NW * k) == 0 and n % NS == 0 and n % BR == 0
    nchunk = e // (NW * k)
    assert nchunk % nbuf == 0

    e4 = edge_index.reshape(2, NW, nchunk, k)
    zeros_h = jnp.zeros((n, h), jnp.float32)
    zeros_3 = jnp.zeros((n, d3), jnp.float32)
    zeros_16 = jnp.zeros((n, 16), jnp.float32)
    ones_16 = jnp.ones((k, 16), jnp.float32)
    W3p = jnp.pad(W3, ((0, 0), (0, d3 - c_out)))
    b1r = b1.reshape(1, h)
    b2r = b2.reshape(1, h)
    b3r = jnp.pad(b3, (0, d3 - c_out)).reshape(1, d3)

    degp = _sc_degree(e4, zeros_16, ones_16, n=n)
    dis2, s1 = _tc_first(degp, x, W1, n=n)
    agg1 = _sc_aggregate(s1, e4, zeros_h, n=n, d=h, nbuf=nbuf)
    s2 = _tc_mid(agg1, s1, dis2, b1r, W2, n=n)
    agg2 = _sc_aggregate(s2, e4, zeros_h, n=n, d=h, nbuf=nbuf)
    s3 = _tc_mid(agg2, s2, dis2, b2r, W3p, n=n)
    agg3 = _sc_aggregate(s3, e4, zeros_3, n=n, d=d3, nbuf=nbuf)
    return _tc_last(agg3, s3, dis2, b3r, n=n, c_out=c_out)
